# baseline (speedup 1.0000x reference)
import functools

import jax
import jax.numpy as jnp
from jax import lax
from jax.experimental import pallas as pl
from jax.experimental.pallas import tpu as pltpu
from jax.experimental.pallas import tpu_sc as plsc

_C = 384
_P = 576
_K = _C // 2
_NCHUNK = 3
_PC = _P // _NCHUNK
_SC_ITERS = 6
_TC_ITERS = 26
_CU = 8
_GB = 4
_SC_IMGS = 10
_SC_ROWS = _SC_IMGS + 1


def _gate_body(w_ref, x_ref, fw_ref, mm_ref):
    for i in range(_GB):
        xb = x_ref[i]
        y = jnp.mean(xb, axis=1, keepdims=True)
        z = jnp.zeros((1, 1), dtype=y.dtype)
        y_prev = jnp.concatenate([z, y[:-1]], axis=0)
        y_next = jnp.concatenate([y[1:], z], axis=0)
        conv = y_prev * w_ref[0] + y * w_ref[1] + y_next * w_ref[2]
        att = jax.nn.sigmoid(conv)
        fw = xb * att
        for j in range(_NCHUNK):
            fw_ref[i, j] = fw[:, j * _PC:(j + 1) * _PC]

        lo = jnp.min(fw, axis=0, keepdims=True)
        hi = jnp.max(fw, axis=0, keepdims=True)
        mm_ref[i, 0:1] = lo - (jnp.abs(lo) * 0.01 + 1e-30)
        mm_ref[i, 1:2] = hi + (jnp.abs(hi) * 0.01 + 1e-30)


def _sc_topk_body(fw_hbm, mm_hbm, out_hbm, buf, mmbuf, obuf):
    wid = lax.axis_index("s") * 2 + lax.axis_index("c")
    img = wid // _NCHUNK
    ch = wid - img * _NCHUNK

    kf = jnp.float32(float(_K))
    inv_k = jnp.float32(1.0 / _K)
    one = jnp.full((16,), 1.0, jnp.float32)
    zero = jnp.zeros((16,), jnp.float32)
    half = jnp.float32(0.5)
    zf = jnp.zeros((16,), jnp.float32)

    pltpu.sync_copy(mm_hbm.at[img], mmbuf)
    pltpu.sync_copy(fw_hbm.at[img, ch], buf)
    base_p = ch * _PC

    def group_body(g, carry):
        sl = pl.ds(g * 16, 16)
        slp = pl.ds(base_p + g * 16, 16)
        lo = mmbuf[0, slp]
        hi = mmbuf[1, slp]

        def bstep(_, lohi):
            lo, hi = lohi
            mid = (lo + hi) * half

            def cs(i, accs):
                c0, c1, c2, c3 = accs
                base = i * _CU
                for u in range(0, _CU, 4):
                    v0 = buf[base + u, sl]
                    v1 = buf[base + u + 1, sl]
                    v2 = buf[base + u + 2, sl]
                    v3 = buf[base + u + 3, sl]
                    c0 = jnp.where(v0 >= mid, c0 + one, c0)
                    c1 = jnp.where(v1 >= mid, c1 + one, c1)
                    c2 = jnp.where(v2 >= mid, c2 + one, c2)
                    c3 = jnp.where(v3 >= mid, c3 + one, c3)
                return c0, c1, c2, c3

            c0, c1, c2, c3 = lax.fori_loop(
                0, _C // _CU, cs, (zero, zero, zero, zero))
            cnt = (c0 + c1) + (c2 + c3)
            pred = cnt >= kf
            return jnp.where(pred, mid, lo), jnp.where(pred, hi, mid)

        lo, hi = lax.fori_loop(0, _SC_ITERS, bstep, (lo, hi))
        t = lo

        def rs(i, accs):
            s0, s1, s2, s3 = accs
            base = i * 4
            s0 = s0 + jnp.maximum(buf[base, sl] - t, 0.0)
            s1 = s1 + jnp.maximum(buf[base + 1, sl] - t, 0.0)
            s2 = s2 + jnp.maximum(buf[base + 2, sl] - t, 0.0)
            s3 = s3 + jnp.maximum(buf[base + 3, sl] - t, 0.0)
            return s0, s1, s2, s3

        s0, s1, s2, s3 = lax.fori_loop(0, _C // 4, rs, (zf, zf, zf, zf))
        obuf[sl] = (((s0 + s1) + (s2 + s3)) + kf * t) * inv_k
        return carry

    lax.fori_loop(0, _PC // 16, group_body, 0)

    pltpu.sync_copy(obuf, out_hbm.at[img, ch])


def _tc_topk_body(fw_ref, mm_ref, out_ref):
    kf = jnp.float32(float(_K))
    inv_k = jnp.float32(1.0 / _K)
    for j in range(_NCHUNK):
        v = fw_ref[0, j]
        lo = mm_ref[0, 0:1, j * _PC:(j + 1) * _PC]
        hi = mm_ref[0, 1:2, j * _PC:(j + 1) * _PC]

        def bstep(_, lohi):
            lo, hi = lohi
            mid = (lo + hi) * 0.5
            cnt = jnp.sum((v >= mid).astype(jnp.float32), axis=0,
                          keepdims=True)
            pred = cnt >= kf
            return jnp.where(pred, mid, lo), jnp.where(pred, hi, mid)

        lo, hi = lax.fori_loop(0, _TC_ITERS, bstep, (lo, hi))
        t = lo
        s = jnp.sum(jnp.maximum(v - t, 0.0), axis=0, keepdims=True)
        out_ref[0, 0, j * _PC:(j + 1) * _PC] = (((s + kf * t) * inv_k))[0]


def kernel(x, w):
    B, C, H, W = x.shape
    P = H * W
    xr = x.reshape(B, C, P)
    fw, mm = pl.pallas_call(
        _gate_body,
        grid=(B // _GB,),
        in_specs=[
            pl.BlockSpec(memory_space=pltpu.SMEM),
            pl.BlockSpec((_GB, C, P), lambda b: (b, 0, 0)),
        ],
        out_specs=[
            pl.BlockSpec((_GB, _NCHUNK, C, _PC), lambda b: (b, 0, 0, 0)),
            pl.BlockSpec((_GB, 2, P), lambda b: (b, 0, 0)),
        ],
        out_shape=[
            jax.ShapeDtypeStruct((B, _NCHUNK, C, _PC), jnp.float32),
            jax.ShapeDtypeStruct((B, 2, P), jnp.float32),
        ],
    )(w, xr)

    mesh = plsc.VectorSubcoreMesh(core_axis_name="c", subcore_axis_name="s")
    sc_topk = functools.partial(
        pl.kernel,
        out_type=jax.ShapeDtypeStruct((_SC_ROWS, _NCHUNK, _PC), jnp.float32),
        mesh=mesh,
        scratch_types=[
            pltpu.VMEM((_C, _PC), jnp.float32),
            pltpu.VMEM((2, _P), jnp.float32),
            pltpu.VMEM((_PC,), jnp.float32),
        ],
    )(_sc_topk_body)
    sc_out = sc_topk(fw, mm)

    n_tc = B - _SC_IMGS
    tc_out = pl.pallas_call(
        _tc_topk_body,
        grid=(n_tc,),
        in_specs=[
            pl.BlockSpec((1, _NCHUNK, C, _PC), lambda b: (b + _SC_IMGS, 0, 0, 0)),
            pl.BlockSpec((1, 2, P), lambda b: (b + _SC_IMGS, 0, 0)),
        ],
        out_specs=pl.BlockSpec((1, 1, P), lambda b: (b, 0, 0)),
        out_shape=jax.ShapeDtypeStruct((n_tc, 1, P), jnp.float32),
    )(fw, mm)

    out = jnp.concatenate(
        [sc_out.reshape(_SC_ROWS, P)[:_SC_IMGS], tc_out.reshape(n_tc, P)],
        axis=0)
    return out.reshape(B, H, W)


# speedup vs baseline: 1.8060x; 1.8060x over previous
import functools

import jax
import jax.numpy as jnp
from jax import lax
from jax.experimental import pallas as pl
from jax.experimental.pallas import tpu as pltpu
from jax.experimental.pallas import tpu_sc as plsc

_C = 384
_P = 576
_K = _C // 2
_NCHUNK = 3
_PC = _P // _NCHUNK
_SC_ITERS = 6
_TC_ITERS = 7
_CU = 8
_GB = 4
_SC_IMGS = 10
_SC_ROWS = _SC_IMGS + 1


def _gate_body(w_ref, x_ref, fw_ref, mm_ref):
    for i in range(_GB):
        xb = x_ref[i]
        y = jnp.mean(xb, axis=1, keepdims=True)
        z = jnp.zeros((1, 1), dtype=y.dtype)
        y_prev = jnp.concatenate([z, y[:-1]], axis=0)
        y_next = jnp.concatenate([y[1:], z], axis=0)
        conv = y_prev * w_ref[0] + y * w_ref[1] + y_next * w_ref[2]
        att = jax.nn.sigmoid(conv)
        fw = xb * att
        for j in range(_NCHUNK):
            fw_ref[i, j] = fw[:, j * _PC:(j + 1) * _PC]

        lo = jnp.min(fw, axis=0, keepdims=True)
        hi = jnp.max(fw, axis=0, keepdims=True)
        mm_ref[i, 0:1] = lo - (jnp.abs(lo) * 0.01 + 1e-30)
        mm_ref[i, 1:2] = hi + (jnp.abs(hi) * 0.01 + 1e-30)


def _sc_topk_body(fw_hbm, mm_hbm, out_hbm, buf, mmbuf, obuf):
    wid = lax.axis_index("s") * 2 + lax.axis_index("c")
    img = wid // _NCHUNK
    ch = wid - img * _NCHUNK

    kf = jnp.float32(float(_K))
    inv_k = jnp.float32(1.0 / _K)
    one = jnp.full((16,), 1.0, jnp.float32)
    zero = jnp.zeros((16,), jnp.float32)
    half = jnp.float32(0.5)
    zf = jnp.zeros((16,), jnp.float32)

    pltpu.sync_copy(mm_hbm.at[img], mmbuf)
    pltpu.sync_copy(fw_hbm.at[img, ch], buf)
    base_p = ch * _PC

    def group_body(g, carry):
        sl = pl.ds(g * 16, 16)
        slp = pl.ds(base_p + g * 16, 16)
        lo = mmbuf[0, slp]
        hi = mmbuf[1, slp]

        def bstep(_, lohi):
            lo, hi = lohi
            mid = (lo + hi) * half

            def cs(i, accs):
                c0, c1, c2, c3 = accs
                base = i * _CU
                for u in range(0, _CU, 4):
                    v0 = buf[base + u, sl]
                    v1 = buf[base + u + 1, sl]
                    v2 = buf[base + u + 2, sl]
                    v3 = buf[base + u + 3, sl]
                    c0 = jnp.where(v0 >= mid, c0 + one, c0)
                    c1 = jnp.where(v1 >= mid, c1 + one, c1)
                    c2 = jnp.where(v2 >= mid, c2 + one, c2)
                    c3 = jnp.where(v3 >= mid, c3 + one, c3)
                return c0, c1, c2, c3

            c0, c1, c2, c3 = lax.fori_loop(
                0, _C // _CU, cs, (zero, zero, zero, zero))
            cnt = (c0 + c1) + (c2 + c3)
            pred = cnt >= kf
            return jnp.where(pred, mid, lo), jnp.where(pred, hi, mid)

        lo, hi = lax.fori_loop(0, _SC_ITERS, bstep, (lo, hi))
        t = lo

        def rs(i, accs):
            s0, s1, s2, s3 = accs
            base = i * 4
            s0 = s0 + jnp.maximum(buf[base, sl] - t, 0.0)
            s1 = s1 + jnp.maximum(buf[base + 1, sl] - t, 0.0)
            s2 = s2 + jnp.maximum(buf[base + 2, sl] - t, 0.0)
            s3 = s3 + jnp.maximum(buf[base + 3, sl] - t, 0.0)
            return s0, s1, s2, s3

        s0, s1, s2, s3 = lax.fori_loop(0, _C // 4, rs, (zf, zf, zf, zf))
        obuf[sl] = (((s0 + s1) + (s2 + s3)) + kf * t) * inv_k
        return carry

    lax.fori_loop(0, _PC // 16, group_body, 0)

    pltpu.sync_copy(obuf, out_hbm.at[img, ch])


def _tc_topk_body(fw_ref, mm_ref, out_ref):
    kf = jnp.float32(float(_K))
    inv_k = jnp.float32(1.0 / _K)
    for j in range(_NCHUNK):
        v = fw_ref[0, j]
        lo = mm_ref[0, 0:1, j * _PC:(j + 1) * _PC]
        hi = mm_ref[0, 1:2, j * _PC:(j + 1) * _PC]

        def bstep(_, lohi):
            lo, hi = lohi
            mid = (lo + hi) * 0.5
            cnt = jnp.sum((v >= mid).astype(jnp.float32), axis=0,
                          keepdims=True)
            pred = cnt >= kf
            return jnp.where(pred, mid, lo), jnp.where(pred, hi, mid)

        lo, hi = lax.fori_loop(0, _TC_ITERS, bstep, (lo, hi))
        t = lo
        s = jnp.sum(jnp.maximum(v - t, 0.0), axis=0, keepdims=True)
        out_ref[0, 0, j * _PC:(j + 1) * _PC] = (((s + kf * t) * inv_k))[0]


def kernel(x, w):
    B, C, H, W = x.shape
    P = H * W
    xr = x.reshape(B, C, P)
    fw, mm = pl.pallas_call(
        _gate_body,
        grid=(B // _GB,),
        in_specs=[
            pl.BlockSpec(memory_space=pltpu.SMEM),
            pl.BlockSpec((_GB, C, P), lambda b: (b, 0, 0)),
        ],
        out_specs=[
            pl.BlockSpec((_GB, _NCHUNK, C, _PC), lambda b: (b, 0, 0, 0)),
            pl.BlockSpec((_GB, 2, P), lambda b: (b, 0, 0)),
        ],
        out_shape=[
            jax.ShapeDtypeStruct((B, _NCHUNK, C, _PC), jnp.float32),
            jax.ShapeDtypeStruct((B, 2, P), jnp.float32),
        ],
    )(w, xr)

    mesh = plsc.VectorSubcoreMesh(core_axis_name="c", subcore_axis_name="s")
    sc_topk = functools.partial(
        pl.kernel,
        out_type=jax.ShapeDtypeStruct((_SC_ROWS, _NCHUNK, _PC), jnp.float32),
        mesh=mesh,
        scratch_types=[
            pltpu.VMEM((_C, _PC), jnp.float32),
            pltpu.VMEM((2, _P), jnp.float32),
            pltpu.VMEM((_PC,), jnp.float32),
        ],
    )(_sc_topk_body)
    sc_out = sc_topk(fw, mm)

    n_tc = B - _SC_IMGS
    tc_out = pl.pallas_call(
        _tc_topk_body,
        grid=(n_tc,),
        in_specs=[
            pl.BlockSpec((1, _NCHUNK, C, _PC), lambda b: (b + _SC_IMGS, 0, 0, 0)),
            pl.BlockSpec((1, 2, P), lambda b: (b + _SC_IMGS, 0, 0)),
        ],
        out_specs=pl.BlockSpec((1, 1, P), lambda b: (b, 0, 0)),
        out_shape=jax.ShapeDtypeStruct((n_tc, 1, P), jnp.float32),
    )(fw, mm)

    out = jnp.concatenate(
        [sc_out.reshape(_SC_ROWS, P)[:_SC_IMGS], tc_out.reshape(n_tc, P)],
        axis=0)
    return out.reshape(B, H, W)
